# baseline (speedup 1.0000x reference)
import functools

import jax
import jax.numpy as jnp
from jax import lax
from jax.experimental import pallas as pl
from jax.experimental.pallas import tpu as pltpu
from jax.experimental.pallas import tpu_sc as plsc

N_TABLES = 26
TABLE_SIZE = 100000
EMBED_DIM = 32
BATCH = 16384
NUM_WORKERS = 32
B_PER_W = BATCH // NUM_WORKERS
NBUF = 3
CB = 32
NUM_CHUNKS = B_PER_W // CB

_mesh = plsc.VectorSubcoreMesh(core_axis_name="c", subcore_axis_name="s")


@functools.partial(
    pl.kernel,
    mesh=_mesh,
    out_type=jax.ShapeDtypeStruct((BATCH * N_TABLES, EMBED_DIM), jnp.float32),
    scratch_types=[
        pltpu.VMEM((B_PER_W, 32), jnp.int32),
        pltpu.VMEM((1, B_PER_W * N_TABLES), jnp.int32),
        pltpu.VMEM((NBUF, CB * N_TABLES, EMBED_DIM), jnp.float32),
        pltpu.SemaphoreType.DMA((NBUF,)),
        pltpu.SemaphoreType.DMA((NBUF,)),
    ],
    compiler_params=pltpu.CompilerParams(use_tc_tiling_on_sc=False),
)
def _embed(idx_hbm, table_hbm, out_hbm, idx_v, idx_f, rows_v, gsem, wsem):
    wid = lax.axis_index("s") * 2 + lax.axis_index("c")
    b0 = wid * B_PER_W

    pltpu.sync_copy(idx_hbm.at[pl.ds(b0, B_PER_W)], idx_v)

    i16 = lax.broadcasted_iota(jnp.int32, (16,), 0)
    off_a = i16 * TABLE_SIZE
    off_b = (i16 + 10) * TABLE_SIZE

    def row_body(r, _):
        idx_f[0, pl.ds(r * N_TABLES, 16)] = idx_v[r, pl.ds(0, 16)] + off_a
        idx_f[0, pl.ds(r * N_TABLES + 10, 16)] = (
            idx_v[r, pl.ds(10, 16)] + off_b
        )
        return 0

    lax.fori_loop(0, B_PER_W, row_body, 0)

    CHUNK = CB * N_TABLES
    out_flat = out_hbm
    row0 = b0 * N_TABLES

    def start_gather(c):
        b = c % NBUF
        pltpu.async_copy(
            table_hbm.at[idx_f.at[0, pl.ds(c * CHUNK, CHUNK)]],
            rows_v.at[b],
            gsem.at[b],
        )

    def wait_gather(c):
        b = c % NBUF
        pltpu.make_async_copy(
            table_hbm.at[idx_f.at[0, pl.ds(c * CHUNK, CHUNK)]],
            rows_v.at[b],
            gsem.at[b],
        ).wait()

    def start_writeback(c):
        b = c % NBUF
        pltpu.async_copy(
            rows_v.at[b],
            out_flat.at[pl.ds(row0 + c * CHUNK, CHUNK)],
            wsem.at[b],
        )

    def wait_writeback(c):
        b = c % NBUF
        pltpu.make_async_copy(
            rows_v.at[b],
            out_flat.at[pl.ds(row0 + c * CHUNK, CHUNK)],
            wsem.at[b],
        ).wait()

    for c in range(NBUF - 1):
        start_gather(c)
    for c in range(NUM_CHUNKS):
        wait_gather(c)
        start_writeback(c)
        nxt = c + NBUF - 1
        if nxt < NUM_CHUNKS:
            if c >= 1:
                wait_writeback(nxt - NBUF)
            start_gather(nxt)
    for c in range(NUM_CHUNKS - NBUF, NUM_CHUNKS):
        wait_writeback(c)


def kernel(indices, embedding_table):
    idx_padded = jnp.pad(indices, ((0, 0), (0, 32 - N_TABLES)))
    out = _embed(idx_padded, embedding_table)
    return out.reshape(BATCH, N_TABLES, EMBED_DIM)
